# baseline (speedup 1.0000x reference)
import functools

import jax
import jax.numpy as jnp
from jax import lax
from jax.experimental import pallas as pl
from jax.experimental.pallas import tpu as pltpu
from jax.experimental.pallas import tpu_sc as plsc

S, D, F, E, K = 2048, 1024, 2048, 8, 2
T = 128
CT = 128
NB = (K * S + E * T) // T
N_PAD = NB * T
NC, NS = 2, 16
NW = NC * NS
TPW = S // NW
CH = 32

_SC_MESH = dict(core_axis_name="c", subcore_axis_name="s",
                num_cores=NC, num_subcores=NS)


def _gate_body(x_ref, wg_ref, pos0_ref, pos1_ref, w0_ref, w1_ref, gid_ref):
    x = x_ref[...]
    wg = wg_ref[...]
    logits = jnp.dot(x, wg, preferred_element_type=jnp.float32)
    ie = lax.broadcasted_iota(jnp.int32, (S, E), 1)
    m0 = jnp.max(logits, axis=-1, keepdims=True)
    e0 = jnp.min(jnp.where(logits >= m0, ie, E), axis=-1, keepdims=True)
    l2 = jnp.where(ie == e0, -jnp.inf, logits)
    m1 = jnp.max(l2, axis=-1, keepdims=True)
    e1 = jnp.min(jnp.where(l2 >= m1, ie, E), axis=-1, keepdims=True)
    t = jnp.exp(m1 - m0)
    w0 = 1.0 / (1.0 + t)
    w1 = t / (1.0 + t)

    oh0 = (ie == e0).astype(jnp.float32)
    oh1 = (ie == e1).astype(jnp.float32)

    ci = lax.broadcasted_iota(jnp.int32, (CT, CT), 0)
    cj = lax.broadcasted_iota(jnp.int32, (CT, CT), 1)
    tri = (cj < ci).astype(jnp.float32)

    def excl_counts(oh):
        rows, off = [], jnp.zeros((1, E), jnp.float32)
        for c in range(S // CT):
            blk = oh[c * CT:(c + 1) * CT, :]
            rows.append(off + jnp.dot(tri, blk, preferred_element_type=jnp.float32))
            off = off + jnp.sum(blk, axis=0, keepdims=True)
        return jnp.concatenate(rows, axis=0), off

    r0, cnt0 = excl_counts(oh0)
    r1, cnt1 = excl_counts(oh1)
    cnt = cnt0 + cnt1
    pc = jnp.floor((cnt + (T - 1)) / T) * T
    starts, s = [], jnp.zeros((1, 1), jnp.float32)
    for e in range(E):
        starts.append(s)
        s = s + pc[:, e:e + 1]
    start = jnp.concatenate(starts, axis=1)

    pos0 = jnp.sum(oh0 * (start + r0), axis=-1, keepdims=True)
    pos1 = jnp.sum(oh1 * (start + cnt0 + r1), axis=-1, keepdims=True)
    pos0_ref[...] = pos0.astype(jnp.int32)
    pos1_ref[...] = pos1.astype(jnp.int32)
    w0_ref[...] = jnp.broadcast_to(w0, (S, 16))
    w1_ref[...] = jnp.broadcast_to(w1, (S, 16))

    bt = (lax.broadcasted_iota(jnp.int32, (1, 128), 1) * T).astype(jnp.float32)
    gid = jnp.zeros((1, 128), jnp.int32)
    seq = jnp.zeros((1, 128), jnp.int32)
    for e in range(1, E):
        gid = gid + (bt >= start[:, e:e + 1]).astype(jnp.int32)
    for e in range(E):
        hit = jnp.logical_and(bt >= start[:, e:e + 1], pc[:, e:e + 1] > 0)
        seq = seq + hit.astype(jnp.int32)
    seq = seq - 1
    nxt = jnp.full((1, 128), -1, jnp.int32)
    for e in range(E - 1, -1, -1):
        take = jnp.logical_and(gid < e, pc[:, e:e + 1] > 0)
        nxt = jnp.where(take, e, nxt)
    seq_prev = jnp.concatenate([seq[:, :1] - 1, seq[:, :-1]], axis=1)
    first = (seq != seq_prev).astype(jnp.int32)
    gid_ref[...] = jnp.concatenate([gid, seq, first, nxt], axis=0)


def _gate(x, w_gate):
    return pl.pallas_call(
        _gate_body,
        out_shape=(
            jax.ShapeDtypeStruct((S, 1), jnp.int32),
            jax.ShapeDtypeStruct((S, 1), jnp.int32),
            jax.ShapeDtypeStruct((S, 16), jnp.float32),
            jax.ShapeDtypeStruct((S, 16), jnp.float32),
            jax.ShapeDtypeStruct((4, 128), jnp.int32),
        ),
    )(x, w_gate)


def _sc_scatter(x, pos0, pos1):
    mesh = plsc.VectorSubcoreMesh(**_SC_MESH)

    @functools.partial(
        pl.kernel,
        out_type=jax.ShapeDtypeStruct((N_PAD, D), jnp.float32),
        mesh=mesh,
        scratch_types=[
            pltpu.VMEM((TPW,), jnp.int32),
            pltpu.VMEM((TPW,), jnp.int32),
            pltpu.VMEM((TPW, D), jnp.float32),
            pltpu.SemaphoreType.DMA,
        ],
    )
    def scatter(x_hbm, p0_hbm, p1_hbm, xs_hbm, idx0_v, idx1_v, rows_v, sem):
        wid = lax.axis_index("s") * NC + lax.axis_index("c")
        base = wid * TPW
        pltpu.sync_copy(p0_hbm.at[pl.ds(base, TPW)], idx0_v)
        pltpu.sync_copy(p1_hbm.at[pl.ds(base, TPW)], idx1_v)
        pltpu.sync_copy(x_hbm.at[pl.ds(base, TPW)], rows_v)
        pltpu.async_copy(rows_v, xs_hbm.at[idx0_v], sem).wait()
        pltpu.async_copy(rows_v, xs_hbm.at[idx1_v], sem).wait()

    return scatter(x, pos0, pos1)


def _moe_body(gid_ref, seq_ref, first_ref, nxt_ref,
              xs_ref, wi0_hbm, wi1_hbm, wo_ref, y_ref,
              w0buf, w1buf, sems):
    i = pl.program_id(0)
    sq = seq_ref[i]
    slot = lax.rem(sq, 2)
    nslot = lax.rem(sq + 1, 2)

    def cp0(e, s):
        return pltpu.make_async_copy(wi0_hbm.at[e], w0buf.at[s], sems.at[s, 0])

    def cp1(e, s):
        return pltpu.make_async_copy(wi1_hbm.at[e], w1buf.at[s], sems.at[s, 1])

    @pl.when(i == 0)
    def _():
        cp0(gid_ref[0], slot).start()
        cp1(gid_ref[0], slot).start()

    isfirst = first_ref[i] == 1

    @pl.when(jnp.logical_and(isfirst, nxt_ref[i] >= 0))
    def _():
        cp0(nxt_ref[i], nslot).start()
        cp1(nxt_ref[i], nslot).start()

    x = xs_ref[...]

    @pl.when(isfirst)
    def _():
        cp0(gid_ref[i], slot).wait()

    a = jnp.dot(x, w0buf[slot], preferred_element_type=jnp.float32)

    @pl.when(isfirst)
    def _():
        cp1(gid_ref[i], slot).wait()

    b = jnp.dot(x, w1buf[slot], preferred_element_type=jnp.float32)
    h = (a * jax.nn.sigmoid(a)) * b
    y_ref[...] = jnp.dot(h, wo_ref[0], preferred_element_type=jnp.float32)


def _grouped_mlp(gid, seq, first, nxt, x_sorted, wi0, wi1, wo):
    return pl.pallas_call(
        _moe_body,
        grid_spec=pltpu.PrefetchScalarGridSpec(
            num_scalar_prefetch=4,
            grid=(NB,),
            in_specs=[
                pl.BlockSpec((T, D), lambda i, g, s, fs, nx: (i, 0)),
                pl.BlockSpec(memory_space=pltpu.HBM),
                pl.BlockSpec(memory_space=pltpu.HBM),
                pl.BlockSpec((1, F, D), lambda i, g, s, fs, nx: (g[i], 0, 0)),
            ],
            out_specs=pl.BlockSpec((T, D), lambda i, g, s, fs, nx: (i, 0)),
            scratch_shapes=[
                pltpu.VMEM((2, D, F), jnp.float32),
                pltpu.VMEM((2, D, F), jnp.float32),
                pltpu.SemaphoreType.DMA((2, 2)),
            ],
        ),
        out_shape=jax.ShapeDtypeStruct((N_PAD, D), jnp.float32),
    )(gid, seq, first, nxt, x_sorted, wi0, wi1, wo)


def _sc_combine(y_sorted, pos0, pos1, w0, w1):
    mesh = plsc.VectorSubcoreMesh(**_SC_MESH)
    L = 16

    @functools.partial(
        pl.kernel,
        out_type=jax.ShapeDtypeStruct((S, D), jnp.float32),
        mesh=mesh,
        scratch_types=[
            pltpu.VMEM((CH,), jnp.int32),
            pltpu.VMEM((CH,), jnp.int32),
            pltpu.VMEM((TPW, L), jnp.float32),
            pltpu.VMEM((TPW, L), jnp.float32),
            pltpu.VMEM((CH, D), jnp.float32),
            pltpu.VMEM((CH, D), jnp.float32),
            pltpu.SemaphoreType.DMA,
        ],
    )
    def combine(y_hbm, p0_hbm, p1_hbm, w0_hbm, w1_hbm, out_hbm,
                idx0_v, idx1_v, w0_v, w1_v, buf0, buf1, sem):
        wid = lax.axis_index("s") * NC + lax.axis_index("c")
        base = wid * TPW
        pltpu.sync_copy(w0_hbm.at[pl.ds(base, TPW)], w0_v)
        pltpu.sync_copy(w1_hbm.at[pl.ds(base, TPW)], w1_v)
        for c in range(TPW // CH):
            cbase = base + c * CH
            pltpu.sync_copy(p0_hbm.at[pl.ds(cbase, CH)], idx0_v)
            pltpu.sync_copy(p1_hbm.at[pl.ds(cbase, CH)], idx1_v)
            pltpu.async_copy(y_hbm.at[idx0_v], buf0, sem).wait()
            pltpu.async_copy(y_hbm.at[idx1_v], buf1, sem).wait()

            def token(t2, _):
                for u in range(2):
                    tt = t2 * 2 + u
                    wa = w0_v[c * CH + tt]
                    wb = w1_v[c * CH + tt]
                    for dd in range(D // L):
                        sl = pl.ds(dd * L, L)
                        buf0[tt, sl] = wa * buf0[tt, sl] + wb * buf1[tt, sl]
                return 0

            lax.fori_loop(0, CH // 2, token, 0)
            pltpu.sync_copy(buf0, out_hbm.at[pl.ds(cbase, CH)])

    return combine(y_sorted, pos0, pos1, w0, w1)


def kernel(inputs, w_gate, wi0, wi1, wo):
    x = inputs.reshape(S, D)
    pos0, pos1, w0, w1, meta = _gate(x, w_gate)
    pos0 = pos0.reshape(S)
    pos1 = pos1.reshape(S)
    gid, seq, first, nxt = meta[0], meta[1], meta[2], meta[3]
    x_sorted = _sc_scatter(x, pos0, pos1)
    y_sorted = _grouped_mlp(gid, seq, first, nxt, x_sorted, wi0, wi1, wo)
    out = _sc_combine(y_sorted, pos0, pos1, w0, w1)
    return out.reshape(1, S, D)


# speedup vs baseline: 1.0198x; 1.0198x over previous
import functools

import jax
import jax.numpy as jnp
from jax import lax
from jax.experimental import pallas as pl
from jax.experimental.pallas import tpu as pltpu
from jax.experimental.pallas import tpu_sc as plsc

S, D, F, E, K = 2048, 1024, 2048, 8, 2
T = 128
CT = 128
NB = (K * S + E * T) // T
N_PAD = NB * T
NC, NS = 2, 16
NW = NC * NS
TPW = S // NW
CH = 16

_SC_MESH = dict(core_axis_name="c", subcore_axis_name="s",
                num_cores=NC, num_subcores=NS)


def _gate_body(x_ref, wg_ref, pos0_ref, pos1_ref, w0_ref, w1_ref, gid_ref):
    x = x_ref[...]
    wg = wg_ref[...]
    logits = jnp.dot(x, wg, preferred_element_type=jnp.float32)
    ie = lax.broadcasted_iota(jnp.int32, (S, E), 1)
    m0 = jnp.max(logits, axis=-1, keepdims=True)
    e0 = jnp.min(jnp.where(logits >= m0, ie, E), axis=-1, keepdims=True)
    l2 = jnp.where(ie == e0, -jnp.inf, logits)
    m1 = jnp.max(l2, axis=-1, keepdims=True)
    e1 = jnp.min(jnp.where(l2 >= m1, ie, E), axis=-1, keepdims=True)
    t = jnp.exp(m1 - m0)
    w0 = 1.0 / (1.0 + t)
    w1 = t / (1.0 + t)

    oh0 = (ie == e0).astype(jnp.float32)
    oh1 = (ie == e1).astype(jnp.float32)

    ci = lax.broadcasted_iota(jnp.int32, (CT, CT), 0)
    cj = lax.broadcasted_iota(jnp.int32, (CT, CT), 1)
    tri = (cj < ci).astype(jnp.float32)

    def excl_counts(oh):
        rows, off = [], jnp.zeros((1, E), jnp.float32)
        for c in range(S // CT):
            blk = oh[c * CT:(c + 1) * CT, :]
            rows.append(off + jnp.dot(tri, blk, preferred_element_type=jnp.float32))
            off = off + jnp.sum(blk, axis=0, keepdims=True)
        return jnp.concatenate(rows, axis=0), off

    r0, cnt0 = excl_counts(oh0)
    r1, cnt1 = excl_counts(oh1)
    cnt = cnt0 + cnt1
    pc = jnp.floor((cnt + (T - 1)) / T) * T
    starts, s = [], jnp.zeros((1, 1), jnp.float32)
    for e in range(E):
        starts.append(s)
        s = s + pc[:, e:e + 1]
    start = jnp.concatenate(starts, axis=1)

    pos0 = jnp.sum(oh0 * (start + r0), axis=-1, keepdims=True)
    pos1 = jnp.sum(oh1 * (start + cnt0 + r1), axis=-1, keepdims=True)
    pos0_ref[...] = pos0.astype(jnp.int32)
    pos1_ref[...] = pos1.astype(jnp.int32)
    w0_ref[...] = jnp.broadcast_to(w0, (S, 16))
    w1_ref[...] = jnp.broadcast_to(w1, (S, 16))

    bt = (lax.broadcasted_iota(jnp.int32, (1, 128), 1) * T).astype(jnp.float32)
    gid = jnp.zeros((1, 128), jnp.int32)
    seq = jnp.zeros((1, 128), jnp.int32)
    for e in range(1, E):
        gid = gid + (bt >= start[:, e:e + 1]).astype(jnp.int32)
    for e in range(E):
        hit = jnp.logical_and(bt >= start[:, e:e + 1], pc[:, e:e + 1] > 0)
        seq = seq + hit.astype(jnp.int32)
    seq = seq - 1
    nxt = jnp.full((1, 128), -1, jnp.int32)
    for e in range(E - 1, -1, -1):
        take = jnp.logical_and(gid < e, pc[:, e:e + 1] > 0)
        nxt = jnp.where(take, e, nxt)
    seq_prev = jnp.concatenate([seq[:, :1] - 1, seq[:, :-1]], axis=1)
    first = (seq != seq_prev).astype(jnp.int32)
    gid_ref[...] = jnp.concatenate([gid, seq, first, nxt], axis=0)


def _gate(x, w_gate):
    return pl.pallas_call(
        _gate_body,
        out_shape=(
            jax.ShapeDtypeStruct((S, 1), jnp.int32),
            jax.ShapeDtypeStruct((S, 1), jnp.int32),
            jax.ShapeDtypeStruct((S, 16), jnp.float32),
            jax.ShapeDtypeStruct((S, 16), jnp.float32),
            jax.ShapeDtypeStruct((4, 128), jnp.int32),
        ),
    )(x, w_gate)


def _sc_scatter(x, pos0, pos1):
    mesh = plsc.VectorSubcoreMesh(**_SC_MESH)

    @functools.partial(
        pl.kernel,
        out_type=jax.ShapeDtypeStruct((N_PAD, D), jnp.float32),
        mesh=mesh,
        scratch_types=[
            pltpu.VMEM((TPW,), jnp.int32),
            pltpu.VMEM((TPW,), jnp.int32),
            pltpu.VMEM((TPW, D), jnp.float32),
            pltpu.SemaphoreType.DMA,
        ],
    )
    def scatter(x_hbm, p0_hbm, p1_hbm, xs_hbm, idx0_v, idx1_v, rows_v, sem):
        wid = lax.axis_index("s") * NC + lax.axis_index("c")
        base = wid * TPW
        pltpu.sync_copy(p0_hbm.at[pl.ds(base, TPW)], idx0_v)
        pltpu.sync_copy(p1_hbm.at[pl.ds(base, TPW)], idx1_v)
        pltpu.sync_copy(x_hbm.at[pl.ds(base, TPW)], rows_v)
        pltpu.async_copy(rows_v, xs_hbm.at[idx0_v], sem).wait()
        pltpu.async_copy(rows_v, xs_hbm.at[idx1_v], sem).wait()

    return scatter(x, pos0, pos1)


def _moe_body(gid_ref, seq_ref, first_ref, nxt_ref,
              xs_ref, wi0_hbm, wi1_hbm, wo_ref, y_ref,
              w0buf, w1buf, sems):
    i = pl.program_id(0)
    sq = seq_ref[i]
    slot = lax.rem(sq, 2)
    nslot = lax.rem(sq + 1, 2)

    def cp0(e, s):
        return pltpu.make_async_copy(wi0_hbm.at[e], w0buf.at[s], sems.at[s, 0])

    def cp1(e, s):
        return pltpu.make_async_copy(wi1_hbm.at[e], w1buf.at[s], sems.at[s, 1])

    @pl.when(i == 0)
    def _():
        cp0(gid_ref[0], slot).start()
        cp1(gid_ref[0], slot).start()

    isfirst = first_ref[i] == 1

    @pl.when(jnp.logical_and(isfirst, nxt_ref[i] >= 0))
    def _():
        cp0(nxt_ref[i], nslot).start()
        cp1(nxt_ref[i], nslot).start()

    x = xs_ref[...]

    @pl.when(isfirst)
    def _():
        cp0(gid_ref[i], slot).wait()

    a = jnp.dot(x, w0buf[slot], preferred_element_type=jnp.float32)

    @pl.when(isfirst)
    def _():
        cp1(gid_ref[i], slot).wait()

    b = jnp.dot(x, w1buf[slot], preferred_element_type=jnp.float32)
    h = (a * jax.nn.sigmoid(a)) * b
    y_ref[...] = jnp.dot(h, wo_ref[0], preferred_element_type=jnp.float32)


def _grouped_mlp(gid, seq, first, nxt, x_sorted, wi0, wi1, wo):
    return pl.pallas_call(
        _moe_body,
        grid_spec=pltpu.PrefetchScalarGridSpec(
            num_scalar_prefetch=4,
            grid=(NB,),
            in_specs=[
                pl.BlockSpec((T, D), lambda i, g, s, fs, nx: (i, 0)),
                pl.BlockSpec(memory_space=pltpu.HBM),
                pl.BlockSpec(memory_space=pltpu.HBM),
                pl.BlockSpec((1, F, D), lambda i, g, s, fs, nx: (g[i], 0, 0)),
            ],
            out_specs=pl.BlockSpec((T, D), lambda i, g, s, fs, nx: (i, 0)),
            scratch_shapes=[
                pltpu.VMEM((2, D, F), jnp.float32),
                pltpu.VMEM((2, D, F), jnp.float32),
                pltpu.SemaphoreType.DMA((2, 2)),
            ],
        ),
        out_shape=jax.ShapeDtypeStruct((N_PAD, D), jnp.float32),
    )(gid, seq, first, nxt, x_sorted, wi0, wi1, wo)


def _sc_combine(y_sorted, pos0, pos1, w0, w1):
    mesh = plsc.VectorSubcoreMesh(**_SC_MESH)
    L = 16

    @functools.partial(
        pl.kernel,
        out_type=jax.ShapeDtypeStruct((S, D), jnp.float32),
        mesh=mesh,
        scratch_types=[
            pltpu.VMEM((2, CH), jnp.int32),
            pltpu.VMEM((2, CH), jnp.int32),
            pltpu.VMEM((TPW, L), jnp.float32),
            pltpu.VMEM((TPW, L), jnp.float32),
            pltpu.VMEM((2, CH, D), jnp.float32),
            pltpu.VMEM((2, CH, D), jnp.float32),
            pltpu.SemaphoreType.DMA((2, 2)),
        ],
    )
    def combine(y_hbm, p0_hbm, p1_hbm, w0_hbm, w1_hbm, out_hbm,
                idx0_v, idx1_v, w0_v, w1_v, buf0, buf1, sems):
        wid = lax.axis_index("s") * NC + lax.axis_index("c")
        base = wid * TPW
        pltpu.sync_copy(w0_hbm.at[pl.ds(base, TPW)], w0_v)
        pltpu.sync_copy(w1_hbm.at[pl.ds(base, TPW)], w1_v)

        def issue(c, s):
            cbase = base + c * CH
            pltpu.sync_copy(p0_hbm.at[pl.ds(cbase, CH)], idx0_v.at[s])
            pltpu.sync_copy(p1_hbm.at[pl.ds(cbase, CH)], idx1_v.at[s])
            g0 = pltpu.async_copy(y_hbm.at[idx0_v.at[s]], buf0.at[s], sems.at[s, 0])
            g1 = pltpu.async_copy(y_hbm.at[idx1_v.at[s]], buf1.at[s], sems.at[s, 1])
            return g0, g1

        NCH = TPW // CH
        cur = issue(0, 0)
        for c in range(NCH):
            s = c % 2
            nn = issue(c + 1, (c + 1) % 2) if c + 1 < NCH else None
            cur[0].wait()
            cur[1].wait()

            def token(tt, _):
                wa = w0_v[c * CH + tt]
                wb = w1_v[c * CH + tt]
                for dd in range(D // L):
                    sl = pl.ds(dd * L, L)
                    buf0[s, tt, sl] = wa * buf0[s, tt, sl] + wb * buf1[s, tt, sl]
                return 0

            lax.fori_loop(0, CH, token, 0)
            pltpu.sync_copy(buf0.at[s], out_hbm.at[pl.ds(base + c * CH, CH)])
            cur = nn

    return combine(y_sorted, pos0, pos1, w0, w1)


def kernel(inputs, w_gate, wi0, wi1, wo):
    x = inputs.reshape(S, D)
    pos0, pos1, w0, w1, meta = _gate(x, w_gate)
    pos0 = pos0.reshape(S)
    pos1 = pos1.reshape(S)
    gid, seq, first, nxt = meta[0], meta[1], meta[2], meta[3]
    x_sorted = _sc_scatter(x, pos0, pos1)
    y_sorted = _grouped_mlp(gid, seq, first, nxt, x_sorted, wi0, wi1, wo)
    out = _sc_combine(y_sorted, pos0, pos1, w0, w1)
    return out.reshape(1, S, D)


# speedup vs baseline: 1.0458x; 1.0255x over previous
import functools

import jax
import jax.numpy as jnp
from jax import lax
from jax.experimental import pallas as pl
from jax.experimental.pallas import tpu as pltpu
from jax.experimental.pallas import tpu_sc as plsc

S, D, F, E, K = 2048, 1024, 2048, 8, 2
T = 128
CT = 128
NB = (K * S + E * T) // T
N_PAD = NB * T
NC, NS = 2, 16
NW = NC * NS
TPW = S // NW
CH = 16

_SC_MESH = dict(core_axis_name="c", subcore_axis_name="s",
                num_cores=NC, num_subcores=NS)


def _gate_body(x_ref, wg_ref, pos0_ref, pos1_ref, w0_ref, w1_ref, gid_ref):
    x = x_ref[...]
    wg = wg_ref[...]
    logits = jnp.dot(x, wg, preferred_element_type=jnp.float32)
    ie = lax.broadcasted_iota(jnp.int32, (S, E), 1)
    m0 = jnp.max(logits, axis=-1, keepdims=True)
    e0 = jnp.min(jnp.where(logits >= m0, ie, E), axis=-1, keepdims=True)
    l2 = jnp.where(ie == e0, -jnp.inf, logits)
    m1 = jnp.max(l2, axis=-1, keepdims=True)
    e1 = jnp.min(jnp.where(l2 >= m1, ie, E), axis=-1, keepdims=True)
    t = jnp.exp(m1 - m0)
    w0 = 1.0 / (1.0 + t)
    w1 = t / (1.0 + t)

    oh0 = (ie == e0).astype(jnp.float32)
    oh1 = (ie == e1).astype(jnp.float32)

    ci = lax.broadcasted_iota(jnp.int32, (CT, CT), 0)
    cj = lax.broadcasted_iota(jnp.int32, (CT, CT), 1)
    tri = (cj < ci).astype(jnp.float32)

    def excl_counts(oh):
        rows, off = [], jnp.zeros((1, E), jnp.float32)
        for c in range(S // CT):
            blk = oh[c * CT:(c + 1) * CT, :]
            rows.append(off + jnp.dot(tri, blk, preferred_element_type=jnp.float32))
            off = off + jnp.sum(blk, axis=0, keepdims=True)
        return jnp.concatenate(rows, axis=0), off

    r0, cnt0 = excl_counts(oh0)
    r1, cnt1 = excl_counts(oh1)
    cnt = cnt0 + cnt1
    pc = jnp.floor((cnt + (T - 1)) / T) * T
    starts, s = [], jnp.zeros((1, 1), jnp.float32)
    for e in range(E):
        starts.append(s)
        s = s + pc[:, e:e + 1]
    start = jnp.concatenate(starts, axis=1)

    pos0 = jnp.sum(oh0 * (start + r0), axis=-1, keepdims=True)
    pos1 = jnp.sum(oh1 * (start + cnt0 + r1), axis=-1, keepdims=True)
    pos0_ref[...] = pos0.astype(jnp.int32)
    pos1_ref[...] = pos1.astype(jnp.int32)
    w0_ref[...] = jnp.broadcast_to(w0, (S, 16))
    w1_ref[...] = jnp.broadcast_to(w1, (S, 16))

    bt = (lax.broadcasted_iota(jnp.int32, (1, 128), 1) * T).astype(jnp.float32)
    gid = jnp.zeros((1, 128), jnp.int32)
    seq = jnp.zeros((1, 128), jnp.int32)
    for e in range(1, E):
        gid = gid + (bt >= start[:, e:e + 1]).astype(jnp.int32)
    for e in range(E):
        hit = jnp.logical_and(bt >= start[:, e:e + 1], pc[:, e:e + 1] > 0)
        seq = seq + hit.astype(jnp.int32)
    seq = seq - 1
    nxt = jnp.full((1, 128), -1, jnp.int32)
    for e in range(E - 1, -1, -1):
        take = jnp.logical_and(gid < e, pc[:, e:e + 1] > 0)
        nxt = jnp.where(take, e, nxt)
    seq_prev = jnp.concatenate([seq[:, :1] - 1, seq[:, :-1]], axis=1)
    first = (seq != seq_prev).astype(jnp.int32)
    gid_ref[...] = jnp.concatenate([gid, seq, first, nxt], axis=0)


def _gate(x, w_gate):
    return pl.pallas_call(
        _gate_body,
        out_shape=(
            jax.ShapeDtypeStruct((S, 1), jnp.int32),
            jax.ShapeDtypeStruct((S, 1), jnp.int32),
            jax.ShapeDtypeStruct((S, 16), jnp.float32),
            jax.ShapeDtypeStruct((S, 16), jnp.float32),
            jax.ShapeDtypeStruct((4, 128), jnp.int32),
        ),
    )(x, w_gate)


def _sc_scatter(x, pos0, pos1):
    mesh = plsc.VectorSubcoreMesh(**_SC_MESH)

    @functools.partial(
        pl.kernel,
        out_type=jax.ShapeDtypeStruct((N_PAD, D), jnp.float32),
        mesh=mesh,
        scratch_types=[
            pltpu.VMEM((TPW,), jnp.int32),
            pltpu.VMEM((TPW,), jnp.int32),
            pltpu.VMEM((TPW, D), jnp.float32),
            pltpu.SemaphoreType.DMA,
        ],
    )
    def scatter(x_hbm, p0_hbm, p1_hbm, xs_hbm, idx0_v, idx1_v, rows_v, sem):
        wid = lax.axis_index("s") * NC + lax.axis_index("c")
        base = wid * TPW
        pltpu.sync_copy(p0_hbm.at[pl.ds(base, TPW)], idx0_v)
        pltpu.sync_copy(p1_hbm.at[pl.ds(base, TPW)], idx1_v)
        pltpu.sync_copy(x_hbm.at[pl.ds(base, TPW)], rows_v)
        pltpu.async_copy(rows_v, xs_hbm.at[idx0_v], sem).wait()
        pltpu.async_copy(rows_v, xs_hbm.at[idx1_v], sem).wait()

    return scatter(x, pos0, pos1)


def _moe_body(gid_ref, seq_ref, first_ref, nxt_ref,
              xs_ref, wi0_hbm, wi1_hbm, wo_ref, y_ref,
              w0buf, w1buf, sems):
    i = pl.program_id(0)
    sq = seq_ref[i]
    slot = lax.rem(sq, 2)
    nslot = lax.rem(sq + 1, 2)

    def cp0(e, s):
        return pltpu.make_async_copy(wi0_hbm.at[e], w0buf.at[s], sems.at[s, 0])

    def cp1(e, s):
        return pltpu.make_async_copy(wi1_hbm.at[e], w1buf.at[s], sems.at[s, 1])

    @pl.when(i == 0)
    def _():
        cp0(gid_ref[0], slot).start()
        cp1(gid_ref[0], slot).start()

    isfirst = first_ref[i] == 1

    @pl.when(jnp.logical_and(isfirst, nxt_ref[i] >= 0))
    def _():
        cp0(nxt_ref[i], nslot).start()
        cp1(nxt_ref[i], nslot).start()

    x = xs_ref[...]

    @pl.when(isfirst)
    def _():
        cp0(gid_ref[i], slot).wait()
        cp1(gid_ref[i], slot).wait()

    FH = F // 2
    y = None
    for fc in range(2):
        sl = pl.ds(fc * FH, FH)
        a = jnp.dot(x, w0buf[slot, :, sl], preferred_element_type=jnp.float32)
        b = jnp.dot(x, w1buf[slot, :, sl], preferred_element_type=jnp.float32)
        h = (a * jax.nn.sigmoid(a)) * b
        yc = jnp.dot(h, wo_ref[0, sl], preferred_element_type=jnp.float32)
        y = yc if y is None else y + yc
    y_ref[...] = y


def _grouped_mlp(gid, seq, first, nxt, x_sorted, wi0, wi1, wo):
    return pl.pallas_call(
        _moe_body,
        grid_spec=pltpu.PrefetchScalarGridSpec(
            num_scalar_prefetch=4,
            grid=(NB,),
            in_specs=[
                pl.BlockSpec((T, D), lambda i, g, s, fs, nx: (i, 0)),
                pl.BlockSpec(memory_space=pltpu.HBM),
                pl.BlockSpec(memory_space=pltpu.HBM),
                pl.BlockSpec((1, F, D), lambda i, g, s, fs, nx: (g[i], 0, 0)),
            ],
            out_specs=pl.BlockSpec((T, D), lambda i, g, s, fs, nx: (i, 0)),
            scratch_shapes=[
                pltpu.VMEM((2, D, F), jnp.float32),
                pltpu.VMEM((2, D, F), jnp.float32),
                pltpu.SemaphoreType.DMA((2, 2)),
            ],
        ),
        out_shape=jax.ShapeDtypeStruct((N_PAD, D), jnp.float32),
    )(gid, seq, first, nxt, x_sorted, wi0, wi1, wo)


def _sc_combine(y_sorted, pos0, pos1, w0, w1):
    mesh = plsc.VectorSubcoreMesh(**_SC_MESH)
    L = 16

    @functools.partial(
        pl.kernel,
        out_type=jax.ShapeDtypeStruct((S, D), jnp.float32),
        mesh=mesh,
        scratch_types=[
            pltpu.VMEM((2, CH), jnp.int32),
            pltpu.VMEM((2, CH), jnp.int32),
            pltpu.VMEM((TPW, L), jnp.float32),
            pltpu.VMEM((TPW, L), jnp.float32),
            pltpu.VMEM((2, CH, D), jnp.float32),
            pltpu.VMEM((2, CH, D), jnp.float32),
            pltpu.SemaphoreType.DMA((2, 2)),
        ],
    )
    def combine(y_hbm, p0_hbm, p1_hbm, w0_hbm, w1_hbm, out_hbm,
                idx0_v, idx1_v, w0_v, w1_v, buf0, buf1, sems):
        wid = lax.axis_index("s") * NC + lax.axis_index("c")
        base = wid * TPW
        pltpu.sync_copy(w0_hbm.at[pl.ds(base, TPW)], w0_v)
        pltpu.sync_copy(w1_hbm.at[pl.ds(base, TPW)], w1_v)

        def issue(c, s):
            cbase = base + c * CH
            pltpu.sync_copy(p0_hbm.at[pl.ds(cbase, CH)], idx0_v.at[s])
            pltpu.sync_copy(p1_hbm.at[pl.ds(cbase, CH)], idx1_v.at[s])
            g0 = pltpu.async_copy(y_hbm.at[idx0_v.at[s]], buf0.at[s], sems.at[s, 0])
            g1 = pltpu.async_copy(y_hbm.at[idx1_v.at[s]], buf1.at[s], sems.at[s, 1])
            return g0, g1

        NCH = TPW // CH
        cur = issue(0, 0)
        for c in range(NCH):
            s = c % 2
            nn = issue(c + 1, (c + 1) % 2) if c + 1 < NCH else None
            cur[0].wait()
            cur[1].wait()

            def token(tt, _):
                wa = w0_v[c * CH + tt]
                wb = w1_v[c * CH + tt]
                for dd in range(D // L):
                    sl = pl.ds(dd * L, L)
                    buf0[s, tt, sl] = wa * buf0[s, tt, sl] + wb * buf1[s, tt, sl]
                return 0

            lax.fori_loop(0, CH, token, 0)
            pltpu.sync_copy(buf0.at[s], out_hbm.at[pl.ds(base + c * CH, CH)])
            cur = nn

    return combine(y_sorted, pos0, pos1, w0, w1)


def kernel(inputs, w_gate, wi0, wi1, wo):
    x = inputs.reshape(S, D)
    pos0, pos1, w0, w1, meta = _gate(x, w_gate)
    pos0 = pos0.reshape(S)
    pos1 = pos1.reshape(S)
    gid, seq, first, nxt = meta[0], meta[1], meta[2], meta[3]
    x_sorted = _sc_scatter(x, pos0, pos1)
    y_sorted = _grouped_mlp(gid, seq, first, nxt, x_sorted, wi0, wi1, wo)
    out = _sc_combine(y_sorted, pos0, pos1, w0, w1)
    return out.reshape(1, S, D)
